# baseline (speedup 1.0000x reference)
import functools

import jax
import jax.numpy as jnp
from jax import lax
from jax.experimental import pallas as pl
from jax.experimental.pallas import tpu as pltpu
from jax.experimental.pallas import tpu_sc as plsc

N = 10000; S = 10; NS = N * S; EC = 400000; EO = 160000
DN = 128; DE = 16; NHID = 128; HOP = 16; H = NHID + HOP; LOUT = 2; LIN = 2

BM = 1024
NSP = 100352
GNS = NSP // BM
NB = 1000
GN = N // NB
RNG = 12544
NP = 8
TRASH = RNG
EB = 128
EPAD = 401408
F32 = jnp.float32


def _mm_bias_act(x_ref, w_ref, b_ref, o_ref, *, act):
    y = jnp.dot(x_ref[...], w_ref[...], preferred_element_type=F32) + b_ref[...]
    o_ref[...] = jax.nn.relu(y) if act else y


def _tc_mm(x, w, b, act, bm):
    m, k = x.shape
    n = w.shape[1]
    return pl.pallas_call(
        functools.partial(_mm_bias_act, act=act),
        grid=(m // bm,),
        in_specs=[pl.BlockSpec((bm, k), lambda i: (i, 0)),
                  pl.BlockSpec((k, n), lambda i: (0, 0)),
                  pl.BlockSpec((1, n), lambda i: (0, 0))],
        out_specs=pl.BlockSpec((bm, n), lambda i: (i, 0)),
        out_shape=jax.ShapeDtypeStruct((m, n), F32),
    )(x, w, b.reshape(1, -1))


def _cat_body(h_ref, p_ref, o_ref):
    o_ref[...] = jnp.concatenate([h_ref[...], p_ref[...]], axis=-1)


def _tc_cat(hrows, hopem):
    return pl.pallas_call(
        _cat_body,
        grid=(GNS,),
        in_specs=[pl.BlockSpec((BM, DN), lambda i: (i, 0)),
                  pl.BlockSpec((BM, HOP), lambda i: (i, 0))],
        out_specs=pl.BlockSpec((BM, H), lambda i: (i, 0)),
        out_shape=jax.ShapeDtypeStruct((NSP, H), F32),
    )(hrows, hopem)


def _mm_stats_body(cx_ref, agg_ref, w_ref, o_ref, p_ref):
    y = jnp.dot(cx_ref[...] + agg_ref[...], w_ref[...], preferred_element_type=F32)
    o_ref[...] = y
    i = pl.program_id(0)
    rows = i * BM + lax.broadcasted_iota(jnp.int32, (BM, 1), 0)
    ym = jnp.where(rows < NS, y, 0.0)
    s = jnp.sum(ym, axis=0, keepdims=True)
    s2 = jnp.sum(ym * ym, axis=0, keepdims=True)
    p_ref[...] = jnp.concatenate([s, s2, jnp.zeros((6, H), F32)], axis=0)[None]


def _tc_mm_stats(cx, agg, w):
    return pl.pallas_call(
        _mm_stats_body,
        grid=(GNS,),
        in_specs=[pl.BlockSpec((BM, H), lambda i: (i, 0)),
                  pl.BlockSpec((BM, H), lambda i: (i, 0)),
                  pl.BlockSpec((H, H), lambda i: (0, 0))],
        out_specs=[pl.BlockSpec((BM, H), lambda i: (i, 0)),
                   pl.BlockSpec((1, 8, H), lambda i: (i, 0, 0))],
        out_shape=[jax.ShapeDtypeStruct((NSP, H), F32),
                   jax.ShapeDtypeStruct((GNS, 8, H), F32)],
    )(cx, agg, w)


def _bn_res_body(y_ref, p_ref, prev_ref, o_ref, *, count):
    ps = p_ref[...]
    s = jnp.sum(ps[:, 0, :], axis=0)
    s2 = jnp.sum(ps[:, 1, :], axis=0)
    mean = s / count
    var = s2 / count - mean * mean
    inv = lax.rsqrt(var + 1e-5)
    o_ref[...] = jax.nn.relu((y_ref[...] - mean) * inv) + prev_ref[...]


def _tc_bn_res(y, parts, prev, count, bm):
    m, d = y.shape
    g = parts.shape[0]
    return pl.pallas_call(
        functools.partial(_bn_res_body, count=count),
        grid=(m // bm,),
        in_specs=[pl.BlockSpec((bm, d), lambda i: (i, 0)),
                  pl.BlockSpec((g, 8, d), lambda i: (0, 0, 0)),
                  pl.BlockSpec((bm, d), lambda i: (i, 0))],
        out_specs=pl.BlockSpec((bm, d), lambda i: (i, 0)),
        out_shape=jax.ShapeDtypeStruct((m, d), F32),
    )(y, parts, prev)


def _pool_pre_body(cx_ref, hop_ref, wo_ref, bo_ref, wsub_ref, bsub_ref,
                   wctx_ref, bctx_ref, wgs_ref, bgs_ref, wgx_ref, bgx_ref,
                   cx2_ref, sub_ref, ctxc_ref):
    cx2 = jnp.dot(cx_ref[...], wo_ref[...], preferred_element_type=F32) + bo_ref[...]
    cx2_ref[...] = cx2
    hop = hop_ref[...]
    gs = jax.nn.sigmoid(jnp.dot(hop, wgs_ref[...], preferred_element_type=F32) + bgs_ref[...])
    gx = jax.nn.sigmoid(jnp.dot(hop, wgx_ref[...], preferred_element_type=F32) + bgx_ref[...])
    sub_ref[...] = jax.nn.relu(jnp.dot(cx2, wsub_ref[...], preferred_element_type=F32) + bsub_ref[...]) * gs
    ctx = jax.nn.relu(jnp.dot(cx2, wctx_ref[...], preferred_element_type=F32) + bctx_ref[...]) * gx
    ctxc_ref[...] = jnp.concatenate(
        [ctx, jnp.ones((BM, 1), F32), jnp.zeros((BM, HOP - 1), F32)], axis=-1)


def _tc_pool_pre(cx, hopem, wo, bo, wsub, bsub, wctx, bctx, wgs, bgs, wgx, bgx):
    full = lambda a, b: pl.BlockSpec((a, b), lambda i: (0, 0))
    return pl.pallas_call(
        _pool_pre_body,
        grid=(GNS,),
        in_specs=[pl.BlockSpec((BM, H), lambda i: (i, 0)),
                  pl.BlockSpec((BM, HOP), lambda i: (i, 0)),
                  full(H, NHID), full(1, NHID), full(NHID, NHID), full(1, NHID),
                  full(NHID, NHID), full(1, NHID), full(HOP, NHID), full(1, NHID),
                  full(HOP, NHID), full(1, NHID)],
        out_specs=[pl.BlockSpec((BM, NHID), lambda i: (i, 0)),
                   pl.BlockSpec((BM, NHID), lambda i: (i, 0)),
                   pl.BlockSpec((BM, H), lambda i: (i, 0))],
        out_shape=[jax.ShapeDtypeStruct((NSP, NHID), F32),
                   jax.ShapeDtypeStruct((NSP, NHID), F32),
                   jax.ShapeDtypeStruct((NSP, H), F32)],
    )(cx, hopem, wo, bo.reshape(1, -1), wsub, bsub.reshape(1, -1),
      wctx, bctx.reshape(1, -1), wgs, bgs.reshape(1, -1), wgx, bgx.reshape(1, -1))


GB = 800


def _group_pool_body(sub_ref, cx2_ref, hop_ref, wgc_ref, bgc_ref,
                     subm_ref, cent_ref):
    gg = lax.broadcasted_iota(jnp.int32, (GB // S, GB), 0)
    jj = lax.broadcasted_iota(jnp.int32, (GB // S, GB), 1)
    pmean = jnp.where(jj // S == gg, 1.0 / S, 0.0)
    psel = jnp.where(jj == gg * S, 1.0, 0.0)
    subm_ref[...] = jnp.dot(pmean, sub_ref[...], preferred_element_type=F32)
    hc = jnp.dot(psel, hop_ref[...], preferred_element_type=F32)
    cc = jnp.dot(psel, cx2_ref[...], preferred_element_type=F32)
    gate = jax.nn.sigmoid(jnp.dot(hc, wgc_ref[...], preferred_element_type=F32) + bgc_ref[...])
    cent_ref[...] = cc * gate


def _tc_group_pool(sub_in, cx2, hopem, wgc, bgc):
    return pl.pallas_call(
        _group_pool_body,
        grid=(NS // GB,),
        in_specs=[pl.BlockSpec((GB, NHID), lambda i: (i, 0)),
                  pl.BlockSpec((GB, NHID), lambda i: (i, 0)),
                  pl.BlockSpec((GB, HOP), lambda i: (i, 0)),
                  pl.BlockSpec((HOP, NHID), lambda i: (0, 0)),
                  pl.BlockSpec((1, NHID), lambda i: (0, 0))],
        out_specs=[pl.BlockSpec((GB // S, NHID), lambda i: (i, 0)),
                   pl.BlockSpec((GB // S, NHID), lambda i: (i, 0))],
        out_shape=[jax.ShapeDtypeStruct((N, NHID), F32),
                   jax.ShapeDtypeStruct((N, NHID), F32)],
    )(sub_in, cx2, hopem, wgc, bgc.reshape(1, -1))


def _combine_body(cent_ref, subm_ref, acc_ref, woe_ref, boe_ref, xk_ref, p_ref):
    a = acc_ref[...]
    csum = a[0, :, :NHID] + a[1, :, :NHID]
    cnt = a[0, :, NHID:NHID + 1] + a[1, :, NHID:NHID + 1]
    ctx = csum / jnp.maximum(cnt, 1.0)
    pooled = cent_ref[...] + subm_ref[...] + ctx
    y = jnp.dot(pooled, woe_ref[...], preferred_element_type=F32) + boe_ref[...]
    xk_ref[...] = y
    s = jnp.sum(y, axis=0, keepdims=True)
    s2 = jnp.sum(y * y, axis=0, keepdims=True)
    p_ref[...] = jnp.concatenate([s, s2, jnp.zeros((6, NHID), F32)], axis=0)[None]


def _tc_combine(cent, subm, acc2, woe, boe):
    return pl.pallas_call(
        _combine_body,
        grid=(GN,),
        in_specs=[pl.BlockSpec((NB, NHID), lambda i: (i, 0)),
                  pl.BlockSpec((NB, NHID), lambda i: (i, 0)),
                  pl.BlockSpec((2, NB, H), lambda i: (0, i, 0)),
                  pl.BlockSpec((NHID, NHID), lambda i: (0, 0)),
                  pl.BlockSpec((1, NHID), lambda i: (0, 0))],
        out_specs=[pl.BlockSpec((NB, NHID), lambda i: (i, 0)),
                   pl.BlockSpec((1, 8, NHID), lambda i: (i, 0, 0))],
        out_shape=[jax.ShapeDtypeStruct((N, NHID), F32),
                   jax.ShapeDtypeStruct((GN, 8, NHID), F32)],
    )(cent, subm, acc2, woe, boe.reshape(1, -1))


def _final_body(h_ref, w1_ref, b1_ref, w2_ref, b2_ref, o_ref):
    y = jax.nn.relu(jnp.dot(h_ref[...], w1_ref[...], preferred_element_type=F32) + b1_ref[...])
    o_ref[...] = jnp.dot(y, w2_ref[...], preferred_element_type=F32) + b2_ref[...]


def _tc_final(h, w1, b1, w2, b2):
    return pl.pallas_call(
        _final_body,
        grid=(GN,),
        in_specs=[pl.BlockSpec((NB, NHID), lambda i: (i, 0)),
                  pl.BlockSpec((NHID, NHID), lambda i: (0, 0)),
                  pl.BlockSpec((1, NHID), lambda i: (0, 0)),
                  pl.BlockSpec((NHID, NHID), lambda i: (0, 0)),
                  pl.BlockSpec((1, NHID), lambda i: (0, 0))],
        out_specs=pl.BlockSpec((NB, NHID), lambda i: (i, 0)),
        out_shape=jax.ShapeDtypeStruct((N, NHID), F32),
    )(h, w1, b1.reshape(1, -1), w2, b2.reshape(1, -1))


def _edge_ranges(dst):
    key = dst // RNG
    pos_in = jnp.zeros((EC,), jnp.int32)
    lens = []
    for p in range(NP):
        m = key == p
        c = jnp.cumsum(m.astype(jnp.int32))
        pos_in = jnp.where(m, c - 1, pos_in)
        lens.append(c[-1])
    lens = jnp.stack(lens)
    plens = ((lens + EB - 1) // EB) * EB
    poff = jnp.concatenate([jnp.zeros((1,), jnp.int32), jnp.cumsum(plens)])[:NP]
    slot = poff[key] + pos_in
    meta = jnp.zeros((16,), jnp.int32).at[:NP].set(plens // EB)
    boff = jnp.zeros((16,), jnp.int32).at[:NP].set(poff // EB)
    dstl = dst - key * RNG
    return slot, meta, boff, dstl, poff


def kernel(x, edge_attr, W_in, b_in, W_edge, b_edge, hop_table, Wc, Wo, bo,
           Wsub, bsub, Wctx, bctx, Wgc, bgc, Wgs, bgs, Wgx, bgx, Woe, boe,
           Wd1, bd1, Wd2, bd2, combined_subgraphs, subgraphs_nodes_mapper,
           subgraphs_edges_mapper, subgraphs_batch, hop_indicator):
    src = combined_subgraphs[0].astype(jnp.int32)
    dst = combined_subgraphs[1].astype(jnp.int32)
    em = subgraphs_edges_mapper.astype(jnp.int32)
    nm = subgraphs_nodes_mapper.astype(jnp.int32)
    hop1 = (hop_indicator + 1).astype(jnp.int32)

    slot, meta, boff, dstl, poff = _edge_ranges(dst)
    srcp = jnp.zeros((EPAD,), jnp.int32).at[slot].set(src)
    emp = jnp.zeros((EPAD,), jnp.int32).at[slot].set(em)
    dstlp = jnp.full((EPAD,), TRASH, jnp.int32).at[slot].set(dstl)
    nm_pad = jnp.concatenate([nm, jnp.full((NSP - NS,), N, jnp.int32)])
    hop_pad = jnp.concatenate([hop1, jnp.zeros((NSP - NS,), jnp.int32)])

    h = _tc_mm(x, W_in, b_in, True, NB)

    for l in range(LOUT):
        ea = _tc_mm(edge_attr, W_edge[l], b_edge[l], True, 2000)

        hrows = jnp.pad(h[nm], ((0, NSP - NS), (0, 0)))
        hopem = hop_table[l][hop_pad]
        cx = _tc_cat(hrows, hopem)
        prev = cx

        for il in range(LIN):
            msg = jax.nn.relu(cx[srcp] + ea[emp])
            pj = jnp.sum(jnp.arange(EPAD, dtype=jnp.int32)[:, None]
                         >= poff[None, :], axis=1) - 1
            gdst = jnp.where(dstlp == TRASH, NSP, dstlp + pj * RNG)
            agg = jax.ops.segment_sum(msg, gdst, num_segments=NSP + 1)[:NSP]

            y, parts = _tc_mm_stats(cx, agg, Wc[l, il])
            cx = _tc_bn_res(y, parts, prev, float(NS), BM)
            prev = cx

        cx2, sub_in, ctxc = _tc_pool_pre(
            cx, hopem, Wo[l], bo[l], Wsub[l], bsub[l], Wctx[l], bctx[l],
            Wgs[l], bgs[l], Wgx[l], bgx[l])

        subm, cent = _tc_group_pool(sub_in[:NS], cx2[:NS], hopem[:NS], Wgc[l], bgc[l])

        half = NSP // 2
        a0 = jax.ops.segment_sum(ctxc[:half], nm_pad[:half], num_segments=N + 16)
        a1 = jax.ops.segment_sum(ctxc[half:], nm_pad[half:], num_segments=N + 16)
        acc2 = jnp.stack([a0, a1])

        xk, parts = _tc_combine(cent, subm, acc2, Woe[l], boe[l])
        h = _tc_bn_res(xk, parts, h, float(N), NB)

    return _tc_final(h, Wd1, bd1, Wd2, bd2)


# speedup vs baseline: 2.3296x; 2.3296x over previous
import functools

import jax
import jax.numpy as jnp
from jax import lax
from jax.experimental import pallas as pl
from jax.experimental.pallas import tpu as pltpu
from jax.experimental.pallas import tpu_sc as plsc

N = 10000; S = 10; NS = N * S; EC = 400000; EO = 160000
DN = 128; DE = 16; NHID = 128; HOP = 16; H = NHID + HOP; LOUT = 2; LIN = 2

HP = 256
BM = 1024
NSP = 100352
GNS = NSP // BM
NB = 1000
GN = N // NB
RNG = 512
NP = 196
TRASH = RNG
EB = 128
EPAD = 425088
F32 = jnp.float32

NWRK = 32
GRB = 112
NSP_W = NSP // NWRK
NBLK = NSP_W // GRB
ACC3 = 10240
A3W = ACC3 // 16
ACC2R = 640
A2W = ACC2R // 16
DRW = RNG // 16
_MESH = plsc.VectorSubcoreMesh(core_axis_name="c", subcore_axis_name="s")


def _mm_bias_act(x_ref, w_ref, b_ref, o_ref, *, act, pad):
    y = jnp.dot(x_ref[...], w_ref[...], preferred_element_type=F32) + b_ref[...]
    if act:
        y = jax.nn.relu(y)
    if pad:
        y = jnp.concatenate([y, jnp.zeros((y.shape[0], pad), F32)], axis=-1)
    o_ref[...] = y


def _tc_mm(x, w, b, act, bm, pad=0):
    m, k = x.shape
    n = w.shape[1]
    return pl.pallas_call(
        functools.partial(_mm_bias_act, act=act, pad=pad),
        grid=(m // bm,),
        in_specs=[pl.BlockSpec((bm, k), lambda i: (i, 0)),
                  pl.BlockSpec((k, n), lambda i: (0, 0)),
                  pl.BlockSpec((1, n), lambda i: (0, 0))],
        out_specs=pl.BlockSpec((bm, n + pad), lambda i: (i, 0)),
        out_shape=jax.ShapeDtypeStruct((m, n + pad), F32),
    )(x, w, b.reshape(1, -1))


def _cat_body(h_ref, ids_ref, t_ref, o_ref, e_ref):
    ids = ids_ref[0, 0, :]
    oh = (ids[:, None] == lax.broadcasted_iota(jnp.int32, (BM, 20), 1)).astype(F32)
    hopem = jnp.dot(oh, t_ref[...], preferred_element_type=F32)
    e_ref[...] = hopem
    o_ref[...] = jnp.concatenate(
        [h_ref[...], hopem, jnp.zeros((BM, HP - H), F32)], axis=-1)


def _tc_cat(hrows, hop_ids3, table):
    return pl.pallas_call(
        _cat_body,
        grid=(GNS,),
        in_specs=[pl.BlockSpec((BM, DN), lambda i: (i, 0)),
                  pl.BlockSpec((1, 1, BM), lambda i: (i, 0, 0)),
                  pl.BlockSpec((20, HOP), lambda i: (0, 0))],
        out_specs=[pl.BlockSpec((BM, HP), lambda i: (i, 0)),
                   pl.BlockSpec((BM, HOP), lambda i: (i, 0))],
        out_shape=[jax.ShapeDtypeStruct((NSP, HP), F32),
                   jax.ShapeDtypeStruct((NSP, HOP), F32)],
    )(hrows, hop_ids3, table)


def _mm_stats_body(cx_ref, aggA_ref, aggB_ref, w_ref, o_ref, p_ref):
    cx = cx_ref[...]
    xa = cx[:, :NHID] + aggA_ref[...]
    xb = cx[:, NHID:H] + aggB_ref[...][:, :H - NHID]
    y = jnp.dot(jnp.concatenate([xa, xb], axis=-1), w_ref[...],
                preferred_element_type=F32)
    o_ref[...] = y
    i = pl.program_id(0)
    rows = i * BM + lax.broadcasted_iota(jnp.int32, (BM, 1), 0)
    ym = jnp.where(rows < NS, y, 0.0)
    s = jnp.sum(ym, axis=0, keepdims=True)
    s2 = jnp.sum(ym * ym, axis=0, keepdims=True)
    p_ref[...] = jnp.concatenate([s, s2, jnp.zeros((6, H), F32)], axis=0)[None]


def _tc_mm_stats(cx, aggA, aggB, w):
    return pl.pallas_call(
        _mm_stats_body,
        grid=(GNS,),
        in_specs=[pl.BlockSpec((BM, HP), lambda i: (i, 0)),
                  pl.BlockSpec((BM, NHID), lambda i: (i, 0)),
                  pl.BlockSpec((BM, NHID), lambda i: (i, 0)),
                  pl.BlockSpec((H, H), lambda i: (0, 0))],
        out_specs=[pl.BlockSpec((BM, H), lambda i: (i, 0)),
                   pl.BlockSpec((1, 8, H), lambda i: (i, 0, 0))],
        out_shape=[jax.ShapeDtypeStruct((NSP, H), F32),
                   jax.ShapeDtypeStruct((GNS, 8, H), F32)],
    )(cx, aggA, aggB, w)


def _bn_res_body(y_ref, p_ref, prev_ref, o_ref, *, count, pad):
    ps = p_ref[...]
    d = y_ref.shape[-1]
    s = jnp.sum(ps[:, 0, :], axis=0)
    s2 = jnp.sum(ps[:, 1, :], axis=0)
    mean = s / count
    var = s2 / count - mean * mean
    inv = lax.rsqrt(var + 1e-5)
    out = jax.nn.relu((y_ref[...] - mean) * inv) + prev_ref[...][:, :d]
    if pad:
        out = jnp.concatenate([out, jnp.zeros((out.shape[0], pad), F32)], axis=-1)
    o_ref[...] = out


def _tc_bn_res(y, parts, prev, count, bm, pad=0):
    m, d = y.shape
    g = parts.shape[0]
    pw = prev.shape[1]
    return pl.pallas_call(
        functools.partial(_bn_res_body, count=count, pad=pad),
        grid=(m // bm,),
        in_specs=[pl.BlockSpec((bm, d), lambda i: (i, 0)),
                  pl.BlockSpec((g, 8, d), lambda i: (0, 0, 0)),
                  pl.BlockSpec((bm, pw), lambda i: (i, 0))],
        out_specs=pl.BlockSpec((bm, d + pad), lambda i: (i, 0)),
        out_shape=jax.ShapeDtypeStruct((m, d + pad), F32),
    )(y, parts, prev)


def _pool_pre_body(cx_ref, hop_ref, wo_ref, bo_ref, wsub_ref, bsub_ref,
                   wctx_ref, bctx_ref, wgs_ref, bgs_ref, wgx_ref, bgx_ref,
                   cx2_ref, sub_ref, ctxc_ref):
    cx2 = jnp.dot(cx_ref[...], wo_ref[...], preferred_element_type=F32) + bo_ref[...]
    cx2_ref[...] = cx2
    hop = hop_ref[...]
    gs = jax.nn.sigmoid(jnp.dot(hop, wgs_ref[...], preferred_element_type=F32) + bgs_ref[...])
    gx = jax.nn.sigmoid(jnp.dot(hop, wgx_ref[...], preferred_element_type=F32) + bgx_ref[...])
    sub_ref[...] = jax.nn.relu(jnp.dot(cx2, wsub_ref[...], preferred_element_type=F32) + bsub_ref[...]) * gs
    ctxc_ref[...] = jax.nn.relu(jnp.dot(cx2, wctx_ref[...], preferred_element_type=F32) + bctx_ref[...]) * gx


def _tc_pool_pre(cx, hopem, wopad, bo, wsub, bsub, wctx, bctx, wgs, bgs, wgx, bgx):
    full = lambda a, b: pl.BlockSpec((a, b), lambda i: (0, 0))
    return pl.pallas_call(
        _pool_pre_body,
        grid=(GNS,),
        in_specs=[pl.BlockSpec((BM, HP), lambda i: (i, 0)),
                  pl.BlockSpec((BM, HOP), lambda i: (i, 0)),
                  full(HP, NHID), full(1, NHID), full(NHID, NHID), full(1, NHID),
                  full(NHID, NHID), full(1, NHID), full(HOP, NHID), full(1, NHID),
                  full(HOP, NHID), full(1, NHID)],
        out_specs=[pl.BlockSpec((BM, NHID), lambda i: (i, 0)),
                   pl.BlockSpec((BM, NHID), lambda i: (i, 0)),
                   pl.BlockSpec((BM, NHID), lambda i: (i, 0))],
        out_shape=[jax.ShapeDtypeStruct((NSP, NHID), F32),
                   jax.ShapeDtypeStruct((NSP, NHID), F32),
                   jax.ShapeDtypeStruct((NSP, NHID), F32)],
    )(cx, hopem, wopad, bo.reshape(1, -1), wsub, bsub.reshape(1, -1),
      wctx, bctx.reshape(1, -1), wgs, bgs.reshape(1, -1), wgx, bgx.reshape(1, -1))


GB = 800


def _group_pool_body(sub_ref, cx2_ref, hop_ref, wgc_ref, bgc_ref,
                     subm_ref, cent_ref):
    gg = lax.broadcasted_iota(jnp.int32, (GB // S, GB), 0)
    jj = lax.broadcasted_iota(jnp.int32, (GB // S, GB), 1)
    pmean = jnp.where(jj // S == gg, 1.0 / S, 0.0)
    psel = jnp.where(jj == gg * S, 1.0, 0.0)
    subm_ref[...] = jnp.dot(pmean, sub_ref[...], preferred_element_type=F32)
    hc = jnp.dot(psel, hop_ref[...], preferred_element_type=F32)
    cc = jnp.dot(psel, cx2_ref[...], preferred_element_type=F32)
    gate = jax.nn.sigmoid(jnp.dot(hc, wgc_ref[...], preferred_element_type=F32) + bgc_ref[...])
    cent_ref[...] = cc * gate


def _tc_group_pool(sub_in, cx2, hopem, wgc, bgc):
    return pl.pallas_call(
        _group_pool_body,
        grid=(NS // GB,),
        in_specs=[pl.BlockSpec((GB, NHID), lambda i: (i, 0)),
                  pl.BlockSpec((GB, NHID), lambda i: (i, 0)),
                  pl.BlockSpec((GB, HOP), lambda i: (i, 0)),
                  pl.BlockSpec((HOP, NHID), lambda i: (0, 0)),
                  pl.BlockSpec((1, NHID), lambda i: (0, 0))],
        out_specs=[pl.BlockSpec((GB // S, NHID), lambda i: (i, 0)),
                   pl.BlockSpec((GB // S, NHID), lambda i: (i, 0))],
        out_shape=[jax.ShapeDtypeStruct((N, NHID), F32),
                   jax.ShapeDtypeStruct((N, NHID), F32)],
    )(sub_in, cx2, hopem, wgc, bgc.reshape(1, -1))


def _combine_body(cent_ref, subm_ref, acc_ref, cnt_ref, woe_ref, boe_ref,
                  xk_ref, p_ref):
    a = acc_ref[...]
    csum = a[0] + a[1]
    ca = cnt_ref[...]
    cnt = (ca[0] + ca[1])[:, 0:1]
    ctx = csum / jnp.maximum(cnt, 1.0)
    pooled = cent_ref[...] + subm_ref[...] + ctx
    y = jnp.dot(pooled, woe_ref[...], preferred_element_type=F32) + boe_ref[...]
    xk_ref[...] = y
    s = jnp.sum(y, axis=0, keepdims=True)
    s2 = jnp.sum(y * y, axis=0, keepdims=True)
    p_ref[...] = jnp.concatenate([s, s2, jnp.zeros((6, NHID), F32)], axis=0)[None]


def _tc_combine(cent, subm, acc2, counts, woe, boe):
    return pl.pallas_call(
        _combine_body,
        grid=(GN,),
        in_specs=[pl.BlockSpec((NB, NHID), lambda i: (i, 0)),
                  pl.BlockSpec((NB, NHID), lambda i: (i, 0)),
                  pl.BlockSpec((2, NB, NHID), lambda i: (0, i, 0)),
                  pl.BlockSpec((2, NB, NHID), lambda i: (0, i, 0)),
                  pl.BlockSpec((NHID, NHID), lambda i: (0, 0)),
                  pl.BlockSpec((1, NHID), lambda i: (0, 0))],
        out_specs=[pl.BlockSpec((NB, NHID), lambda i: (i, 0)),
                   pl.BlockSpec((1, 8, NHID), lambda i: (i, 0, 0))],
        out_shape=[jax.ShapeDtypeStruct((N, NHID), F32),
                   jax.ShapeDtypeStruct((GN, 8, NHID), F32)],
    )(cent, subm, acc2, counts, woe, boe.reshape(1, -1))


def _final_body(h_ref, w1_ref, b1_ref, w2_ref, b2_ref, o_ref):
    y = jax.nn.relu(jnp.dot(h_ref[...], w1_ref[...], preferred_element_type=F32) + b1_ref[...])
    o_ref[...] = jnp.dot(y, w2_ref[...], preferred_element_type=F32) + b2_ref[...]


def _tc_final(h, w1, b1, w2, b2):
    return pl.pallas_call(
        _final_body,
        grid=(GN,),
        in_specs=[pl.BlockSpec((NB, NHID), lambda i: (i, 0)),
                  pl.BlockSpec((NHID, NHID), lambda i: (0, 0)),
                  pl.BlockSpec((1, NHID), lambda i: (0, 0)),
                  pl.BlockSpec((NHID, NHID), lambda i: (0, 0)),
                  pl.BlockSpec((1, NHID), lambda i: (0, 0))],
        out_specs=pl.BlockSpec((NB, NHID), lambda i: (i, 0)),
        out_shape=jax.ShapeDtypeStruct((N, NHID), F32),
    )(h, w1, b1.reshape(1, -1), w2, b2.reshape(1, -1))


def _zero_vmem(ref, rows, width):
    def body(r, _):
        for ch in range(width // 16):
            ref[r, pl.ds(ch * 16, 16)] = jnp.zeros((16,), F32)
        return 0
    lax.fori_loop(0, rows, body, 0)


def _sc_gather(h, nm0):
    @functools.partial(
        pl.kernel, mesh=_MESH,
        out_type=jax.ShapeDtypeStruct((NSP, DN), F32),
        scratch_types=[pltpu.VMEM((GRB,), jnp.int32),
                       pltpu.VMEM((GRB, DN), F32),
                       pltpu.SemaphoreType.DMA],
    )
    def k(h_hbm, nm_hbm, oh_hbm, idx1, buf1, sem1):
        c = lax.axis_index("c")
        s = lax.axis_index("s")
        w = c * 16 + s

        def body(j, _):
            base = w * NSP_W + j * GRB
            pltpu.sync_copy(nm_hbm.at[pl.ds(base, GRB)], idx1)
            pltpu.async_copy(h_hbm.at[idx1], buf1, sem1).wait()
            pltpu.sync_copy(buf1, oh_hbm.at[pl.ds(base, GRB)])
            return 0

        lax.fori_loop(0, NBLK, body, 0)

    return k(h, nm0)


def _sc_counts(nmT):
    def fill_ones(ref, rows):
        def body(r, _):
            for ch in range(NHID // 16):
                ref[r, pl.ds(ch * 16, 16)] = jnp.ones((16,), F32)
            return 0
        lax.fori_loop(0, rows, body, 0)

    @functools.partial(
        pl.kernel, mesh=_MESH,
        out_type=jax.ShapeDtypeStruct((2, ACC3, NHID), F32),
        scratch_types=[pltpu.VMEM((GRB,), jnp.int32),
                       pltpu.VMEM((GRB, NHID), F32),
                       pltpu.VMEM((128, NHID), F32),
                       pltpu.VMEM_SHARED((ACC3, NHID), F32)],
    )
    def k(nm_hbm, out_hbm, idx, vbuf, zbuf, acc):
        c = lax.axis_index("c")
        s = lax.axis_index("s")
        w = c * 16 + s
        _zero_vmem(zbuf, 128, NHID)
        fill_ones(vbuf, GRB)
        for t in range(A3W // 128):
            pltpu.sync_copy(zbuf, acc.at[pl.ds(s * A3W + t * 128, 128)])
        plsc.subcore_barrier()

        def body(j, _):
            base = w * NSP_W + j * GRB
            pltpu.sync_copy(nm_hbm.at[pl.ds(base, GRB)], idx)
            pltpu.sync_copy(vbuf, acc.at[idx], add=True)
            return 0

        lax.fori_loop(0, NBLK, body, 0)
        plsc.subcore_barrier()
        pltpu.sync_copy(acc.at[pl.ds(s * A3W, A3W)],
                        out_hbm.at[c].at[pl.ds(s * A3W, A3W)])

    return k(nmT)


def _sc_ctx_scatter(ctxc, nmT):
    @functools.partial(
        pl.kernel, mesh=_MESH,
        out_type=jax.ShapeDtypeStruct((2, ACC3, NHID), F32),
        scratch_types=[pltpu.VMEM((GRB,), jnp.int32),
                       pltpu.VMEM((GRB, NHID), F32),
                       pltpu.VMEM((128, NHID), F32),
                       pltpu.VMEM_SHARED((ACC3, NHID), F32)],
    )
    def k(ctx_hbm, nm_hbm, out_hbm, idx, vbuf, zbuf, acc):
        c = lax.axis_index("c")
        s = lax.axis_index("s")
        w = c * 16 + s
        _zero_vmem(zbuf, 128, NHID)
        for t in range(A3W // 128):
            pltpu.sync_copy(zbuf, acc.at[pl.ds(s * A3W + t * 128, 128)])
        plsc.subcore_barrier()

        def body(j, _):
            base = w * NSP_W + j * GRB
            pltpu.sync_copy(nm_hbm.at[pl.ds(base, GRB)], idx)
            pltpu.sync_copy(ctx_hbm.at[pl.ds(base, GRB)], vbuf)
            pltpu.sync_copy(vbuf, acc.at[idx], add=True)
            return 0

        lax.fori_loop(0, NBLK, body, 0)
        plsc.subcore_barrier()
        pltpu.sync_copy(acc.at[pl.ds(s * A3W, A3W)],
                        out_hbm.at[c].at[pl.ds(s * A3W, A3W)])

    return k(ctxc, nmT)


def _sc_message(cx, ea, pk, emp, meta_x):
    @functools.partial(
        pl.kernel, mesh=_MESH,
        out_type=[jax.ShapeDtypeStruct((NSP, NHID), F32),
                  jax.ShapeDtypeStruct((NSP, NHID), F32)],
        scratch_types=[pltpu.VMEM((16,), jnp.int32),
                       pltpu.VMEM((EB,), jnp.int32),
                       pltpu.VMEM((EB,), jnp.int32),
                       pltpu.VMEM((EB,), jnp.int32),
                       pltpu.VMEM((EB,), jnp.int32),
                       pltpu.VMEM((EB, HP), F32),
                       pltpu.VMEM((EB, HP), F32),
                       pltpu.VMEM((EB, NHID), F32),
                       pltpu.VMEM((EB, NHID), F32),
                       pltpu.VMEM((128, NHID), F32),
                       pltpu.VMEM_SHARED((ACC2R, NHID), F32),
                       pltpu.VMEM_SHARED((ACC2R, NHID), F32),
                       pltpu.SemaphoreType.DMA,
                       pltpu.SemaphoreType.DMA],
    )
    def k(cx_hbm, ea_hbm, pk_hbm, em_hbm, mx_hbm,
          aggA_hbm, aggB_hbm, metav, pidx, sidx, eidx, didx, cxb, eab,
          pA, pB, zbuf, accA, accB, sem1, sem2):
        c = lax.axis_index("c")
        s = lax.axis_index("s")
        _zero_vmem(zbuf, 128, NHID)
        _zero_vmem(pB, EB, NHID)

        def range_body(pi, _):
            p = c + 2 * pi
            pltpu.sync_copy(mx_hbm.at[pl.ds(8 * p, 16)], metav)
            mv = metav[...]
            nb = mv[0]
            b0 = mv[1]
            for acc in (accA, accB):
                pltpu.sync_copy(zbuf.at[pl.ds(0, A2W)],
                                acc.at[pl.ds(s * A2W, A2W)])
            plsc.subcore_barrier()

            nj = jnp.maximum((nb - s + 15) // 16, 0)

            def body(j, _):
                eoff = (b0 + s + 16 * j) * EB
                pltpu.sync_copy(pk_hbm.at[pl.ds(eoff, EB)], pidx)
                pltpu.sync_copy(em_hbm.at[pl.ds(eoff, EB)], eidx)
                for ch in range(EB // 16):
                    v = pidx[pl.ds(ch * 16, 16)]
                    sidx[pl.ds(ch * 16, 16)] = v & 131071
                    didx[pl.ds(ch * 16, 16)] = lax.shift_right_logical(v, 17)
                cp1 = pltpu.async_copy(cx_hbm.at[sidx], cxb, sem1)
                cp2 = pltpu.async_copy(ea_hbm.at[eidx], eab, sem2)
                cp1.wait()
                cp2.wait()

                def row(r, _):
                    for ch in range(NHID // 16):
                        a = cxb[r, pl.ds(ch * 16, 16)]
                        b = eab[r, pl.ds(ch * 16, 16)]
                        pA[r, pl.ds(ch * 16, 16)] = jnp.maximum(a + b, 0.0)
                    a = cxb[r, pl.ds(NHID, 16)]
                    b = eab[r, pl.ds(NHID, 16)]
                    pB[r, pl.ds(0, 16)] = jnp.maximum(a + b, 0.0)
                    return 0

                lax.fori_loop(0, EB, row, 0)
                pltpu.sync_copy(pA, accA.at[didx], add=True)
                pltpu.sync_copy(pB, accB.at[didx], add=True)
                return 0

            lax.fori_loop(0, nj, body, 0)
            plsc.subcore_barrier()
            pltpu.sync_copy(accA.at[pl.ds(s * DRW, DRW)],
                            aggA_hbm.at[pl.ds(p * RNG + s * DRW, DRW)])
            pltpu.sync_copy(accB.at[pl.ds(s * DRW, DRW)],
                            aggB_hbm.at[pl.ds(p * RNG + s * DRW, DRW)])
            plsc.subcore_barrier()
            return 0

        lax.fori_loop(0, NP // 2, range_body, 0)

    return k(cx, ea, pk, emp, meta_x)


def _edge_ranges(dst):
    key = dst // RNG
    pos_in = jnp.zeros((EC,), jnp.int32)
    lens = []
    for p in range(NP):
        m = key == p
        c = jnp.cumsum(m.astype(jnp.int32))
        pos_in = jnp.where(m, c - 1, pos_in)
        lens.append(c[-1])
    lens = jnp.stack(lens)
    plens = ((lens + EB - 1) // EB) * EB
    poff = jnp.concatenate([jnp.zeros((1,), jnp.int32), jnp.cumsum(plens)])[:NP]
    slot = poff[key] + pos_in
    pr = 8 * jnp.arange(NP, dtype=jnp.int32)
    meta_x = (jnp.zeros((2048,), jnp.int32)
              .at[pr].set(plens // EB)
              .at[pr + 1].set(poff // EB))
    dstl = dst - key * RNG
    return slot, meta_x, dstl


def kernel(x, edge_attr, W_in, b_in, W_edge, b_edge, hop_table, Wc, Wo, bo,
           Wsub, bsub, Wctx, bctx, Wgc, bgc, Wgs, bgs, Wgx, bgx, Woe, boe,
           Wd1, bd1, Wd2, bd2, combined_subgraphs, subgraphs_nodes_mapper,
           subgraphs_edges_mapper, subgraphs_batch, hop_indicator):
    src = combined_subgraphs[0].astype(jnp.int32)
    dst = combined_subgraphs[1].astype(jnp.int32)
    em = subgraphs_edges_mapper.astype(jnp.int32)
    nm = subgraphs_nodes_mapper.astype(jnp.int32)
    hop1 = (hop_indicator + 1).astype(jnp.int32)

    slot, meta_x, dstl = _edge_ranges(dst)
    pk = (jnp.full((EPAD,), TRASH << 17, jnp.int32)
          .at[slot].set(src | (dstl << 17)))
    emp = jnp.zeros((EPAD,), jnp.int32).at[slot].set(em)
    nm0 = jnp.concatenate([nm, jnp.zeros((NSP - NS,), jnp.int32)])
    nmT = jnp.concatenate([nm, jnp.full((NSP - NS,), N, jnp.int32)])
    hop_pad = jnp.concatenate([hop1, jnp.zeros((NSP - NS,), jnp.int32)])
    hop_ids3 = hop_pad.reshape(GNS, 1, BM)
    zpad2 = jnp.zeros((HP - H, NHID), F32)

    counts = _sc_counts(nmT)
    h = _tc_mm(x, W_in, b_in, True, NB)

    for l in range(LOUT):
        ea = _tc_mm(edge_attr, W_edge[l], b_edge[l], True, 2000, pad=HP - H)

        hrows = _sc_gather(h, nm0)
        cx, hopem = _tc_cat(hrows, hop_ids3, hop_table[l])
        prev = cx

        for il in range(LIN):
            aggA, aggB = _sc_message(cx, ea, pk, emp, meta_x)
            y, parts = _tc_mm_stats(cx, aggA, aggB, Wc[l, il])
            cx = _tc_bn_res(y, parts, prev, float(NS), BM, pad=HP - H)
            prev = cx

        wop = jnp.concatenate([Wo[l], zpad2], axis=0)
        cx2, sub_in, ctxc = _tc_pool_pre(
            cx, hopem, wop, bo[l], Wsub[l], bsub[l], Wctx[l], bctx[l],
            Wgs[l], bgs[l], Wgx[l], bgx[l])

        subm, cent = _tc_group_pool(sub_in[:NS], cx2[:NS], hopem[:NS], Wgc[l], bgc[l])

        acc2 = _sc_ctx_scatter(ctxc, nmT)

        xk, parts = _tc_combine(cent, subm, acc2, counts, Woe[l], boe[l])
        h = _tc_bn_res(xk, parts, h, float(N), NB)

    return _tc_final(h, Wd1, bd1, Wd2, bd2)
